# baseline (speedup 1.0000x reference)
import functools
import math

import jax
import jax.numpy as jnp
import numpy as np
from jax import lax
from jax.experimental import pallas as pl
from jax.experimental.pallas import tpu as pltpu
from jax.experimental.pallas import tpu_sc as plsc

N, E, F_IN, H, C = 10000, 320000, 128, 128, 40
CP = 48
NC, NS = 2, 16
NW = NC * NS
CHUNK = 96
BLK = 15
NB = 7
CH = NB * BLK
DEPTH = 3
E_PAD = NW * CH * CHUNK
NPAD = 128
NACC = 10240

_mesh = plsc.VectorSubcoreMesh(core_axis_name="c", subcore_axis_name="s")


def _make_sc_conv(hf, tc_tiling=True):
    @functools.partial(
        pl.kernel,
        out_type=jax.ShapeDtypeStruct((NC, NACC, hf), jnp.float32),
        mesh=_mesh,
        compiler_params=pltpu.CompilerParams(use_tc_tiling_on_sc=tc_tiling),
        scratch_types=[
            [pltpu.VMEM((BLK, CHUNK), jnp.int32)] * 2,
            [pltpu.VMEM((BLK, CHUNK), jnp.int32)] * 2,
            [pltpu.VMEM((CHUNK, hf), jnp.float32)] * DEPTH,
            [pltpu.SemaphoreType.DMA] * DEPTH,
            [pltpu.SemaphoreType.DMA] * DEPTH,
            pltpu.SemaphoreType.DMA,
            pltpu.SemaphoreType.DMA,
            pltpu.SemaphoreType.DMA,
            pltpu.VMEM_SHARED((NACC, hf), jnp.float32),
        ],
    )
    def conv(table, srcp, dstp, zeros, out,
             idx_s, idx_d, bufs, gsems, ssems, isem, xs, xd, accum):
        c = lax.axis_index("c")
        s = lax.axis_index("s")
        wid = c * NS + s
        rows_per = NACC // NS
        base = s * rows_per
        init = pltpu.async_copy(zeros.at[pl.ds(base, rows_per)],
                                accum.at[pl.ds(base, rows_per)], isem)

        def stage_async(bi, st):
            pltpu.async_copy(srcp.at[wid, bi], idx_s[st], xs)
            pltpu.async_copy(dstp.at[wid, bi], idx_d[st], xd)

        def stage_wait(bi, st):
            pltpu.make_async_copy(srcp.at[wid, bi], idx_s[st], xs).wait()
            pltpu.make_async_copy(dstp.at[wid, bi], idx_d[st], xd).wait()

        def start_g(slot, st, b):
            pltpu.async_copy(table.at[idx_s[st].at[slot]], bufs[b], gsems[b])

        def wait_g(slot, st, b):
            pltpu.make_async_copy(table.at[idx_s[st].at[slot]], bufs[b],
                                  gsems[b]).wait()

        def start_s(slot, st, b):
            pltpu.async_copy(bufs[b], accum.at[idx_d[st].at[slot]], ssems[b],
                             add=True)

        def wait_s(slot, st, b):
            pltpu.make_async_copy(bufs[b], accum.at[idx_d[st].at[slot]],
                                  ssems[b]).wait()

        stage_async(0, 0)
        stage_wait(0, 0)
        stage_async(1, 1)
        for b in range(DEPTH):
            start_g(b, 0, b)
        init.wait()
        plsc.subcore_barrier()

        for k in range(CH):
            blk, slot, st, b = k // BLK, k % BLK, (k // BLK) % 2, k % DEPTH
            wait_g(slot, st, b)
            start_s(slot, st, b)
            wait_s(slot, st, b)
            if slot == BLK - 1 and blk + 2 < NB:
                stage_async(blk + 2, blk % 2)
            kn = k + DEPTH
            if kn < CH:
                bn, sn, stn = kn // BLK, kn % BLK, (kn // BLK) % 2
                if bn != blk and sn == 0:
                    stage_wait(bn, stn)
                start_g(sn, stn, b)

        plsc.subcore_barrier()
        pltpu.sync_copy(accum.at[pl.ds(base, rows_per)],
                        out.at[c, pl.ds(base, rows_per)])

    return conv


_sc_conv_h = _make_sc_conv(H)
_sc_conv_c = _make_sc_conv(CP, tc_tiling=False)


@functools.partial(
    pl.kernel,
    out_type=jax.ShapeDtypeStruct((NC, NACC), jnp.float32),
    mesh=_mesh,
    scratch_types=[
        pltpu.VMEM((BLK, CHUNK), jnp.int32),
        pltpu.VMEM((CHUNK,), jnp.float32),
        pltpu.SemaphoreType.DMA,
        pltpu.VMEM_SHARED((NACC,), jnp.float32),
    ],
)
def _sc_degree(dstp, zeros, out, idx_d, ones, ssem, accum):
    c = lax.axis_index("c")
    s = lax.axis_index("s")
    wid = c * NS + s
    rows_per = NACC // NS
    base = s * rows_per
    pltpu.sync_copy(zeros.at[pl.ds(base, rows_per)],
                    accum.at[pl.ds(base, rows_per)])

    @pl.loop(0, CHUNK // 16)
    def _(i):
        ones[pl.ds(i * 16, 16)] = jnp.ones((16,), jnp.float32)

    plsc.subcore_barrier()

    @pl.loop(0, NB)
    def _(bi):
        pltpu.sync_copy(dstp.at[wid, bi], idx_d)
        for k in range(BLK):
            pltpu.async_copy(ones, accum.at[idx_d.at[k]], ssem, add=True)
        for k in range(BLK):
            pltpu.make_async_copy(ones, accum.at[idx_d.at[k]], ssem).wait()

    plsc.subcore_barrier()
    pltpu.sync_copy(accum.at[pl.ds(base, rows_per)],
                    out.at[c, pl.ds(base, rows_per)])


_BN_SCALE = np.float32(1.0 / math.sqrt(1.0 + 1e-5))


def _tc0_body(x, w, out):
    out[...] = jnp.dot(x[...], w[...], preferred_element_type=jnp.float32)


_tc0 = pl.pallas_call(
    _tc0_body, out_shape=jax.ShapeDtypeStruct((N, H), jnp.float32))


def _tc1_body(degp, t, dis, table):
    deg = 1.0 + degp[0][:N] + degp[1][:N]
    d = lax.rsqrt(deg)
    dis[...] = d
    table[...] = t[...] * d


_tc1 = pl.pallas_call(
    _tc1_body,
    out_shape=(jax.ShapeDtypeStruct((N, 1), jnp.float32),
               jax.ShapeDtypeStruct((N, H), jnp.float32)),
)


def _tc_mid_body(p, table, dis, b, g, be, w, out):
    acc = p[0][:N] + p[1][:N] + table[...]
    d = dis[...]
    pre = d * acc + b[...]
    h = jnp.maximum(g[...] * (pre * _BN_SCALE) + be[...], 0.0)
    out[...] = jnp.dot(h, w[...], preferred_element_type=jnp.float32) * d


_tc2 = pl.pallas_call(
    _tc_mid_body, out_shape=jax.ShapeDtypeStruct((N, H), jnp.float32))
_tc3 = pl.pallas_call(
    _tc_mid_body, out_shape=jax.ShapeDtypeStruct((N, CP), jnp.float32))


def _tc4_body(p, table, dis, b, out):
    acc = p[0][:N] + p[1][:N] + table[...]
    pre = dis[...] * acc + b[...]
    col = lax.broadcasted_iota(jnp.int32, (N, CP), 1)
    masked = jnp.where(col < C, pre, -jnp.inf)
    m = jnp.max(masked, axis=1, keepdims=True)
    ex = jnp.exp(masked - m)
    res = pre - m - jnp.log(jnp.sum(ex, axis=1, keepdims=True))
    out[...] = res[:, :C]


_tc4 = pl.pallas_call(
    _tc4_body, out_shape=jax.ShapeDtypeStruct((N, C), jnp.float32))


def kernel(x, edge_index, W1, b1, g1, be1, W2, b2, g2, be2, W3, b3):
    src = edge_index[0]
    dst = edge_index[1]
    pad = E_PAD - E
    pad_idx = jnp.arange(pad, dtype=jnp.int32) % NPAD
    srcp = jnp.concatenate([src, pad_idx]).reshape(NW, NB, BLK, CHUNK)
    dstp = jnp.concatenate([dst, N + pad_idx]).reshape(NW, NB, BLK, CHUNK)
    z1 = jnp.zeros((NACC,), jnp.float32)
    zh = jnp.zeros((NACC, H), jnp.float32)
    zc = jnp.zeros((NACC, CP), jnp.float32)

    t1raw = _tc0(x, W1)
    degp = _sc_degree(dstp, z1)
    dis, t1 = _tc1(degp.reshape(NC, NACC, 1), t1raw)
    p1 = _sc_conv_h(t1, srcp, dstp, zh)
    t2 = _tc2(p1, t1, dis, b1.reshape(1, H), g1.reshape(1, H),
              be1.reshape(1, H), W2)
    p2 = _sc_conv_h(t2, srcp, dstp, zh)
    w3p = jnp.pad(W3, ((0, 0), (0, CP - C)))
    b3p = jnp.pad(b3, (0, CP - C))
    t3 = _tc3(p2, t2, dis, b2.reshape(1, H), g2.reshape(1, H),
              be2.reshape(1, H), w3p)
    p3 = _sc_conv_c(t3, srcp, dstp, zc)
    return _tc4(p3, t3, dis, b3p.reshape(1, CP))


# speedup vs baseline: 1.0035x; 1.0035x over previous
import functools
import math

import jax
import jax.numpy as jnp
import numpy as np
from jax import lax
from jax.experimental import pallas as pl
from jax.experimental.pallas import tpu as pltpu
from jax.experimental.pallas import tpu_sc as plsc

N, E, F_IN, H, C = 10000, 320000, 128, 128, 40
CP = 48
NC, NS = 2, 16
NW = NC * NS
CHUNK = 96
BLK = 15
NB = 7
CH = NB * BLK
DEPTH = 3
E_PAD = NW * CH * CHUNK
NPAD = 128
NACC = 10240

_mesh = plsc.VectorSubcoreMesh(core_axis_name="c", subcore_axis_name="s")


def _make_sc_conv(hf, tc_tiling=True):
    @functools.partial(
        pl.kernel,
        out_type=jax.ShapeDtypeStruct((NC, NACC, hf), jnp.float32),
        mesh=_mesh,
        compiler_params=pltpu.CompilerParams(use_tc_tiling_on_sc=tc_tiling),
        scratch_types=[
            [pltpu.VMEM((BLK, CHUNK), jnp.int32)] * 2,
            [pltpu.VMEM((BLK, CHUNK), jnp.int32)] * 2,
            [pltpu.VMEM((CHUNK, hf), jnp.float32)] * DEPTH,
            [pltpu.SemaphoreType.DMA] * DEPTH,
            [pltpu.SemaphoreType.DMA] * DEPTH,
            pltpu.SemaphoreType.DMA,
            pltpu.SemaphoreType.DMA,
            pltpu.SemaphoreType.DMA,
            pltpu.VMEM_SHARED((NACC, hf), jnp.float32),
        ],
    )
    def conv(table, srcp, dstp, zeros, out,
             idx_s, idx_d, bufs, gsems, ssems, isem, xs, xd, accum):
        c = lax.axis_index("c")
        s = lax.axis_index("s")
        wid = c * NS + s
        rows_per = NACC // NS
        base = s * rows_per
        init = pltpu.async_copy(zeros.at[pl.ds(base, rows_per)],
                                accum.at[pl.ds(base, rows_per)], isem)

        def stage_async(bi, st):
            pltpu.async_copy(srcp.at[wid, bi], idx_s[st], xs)
            pltpu.async_copy(dstp.at[wid, bi], idx_d[st], xd)

        def stage_wait(bi, st):
            pltpu.make_async_copy(srcp.at[wid, bi], idx_s[st], xs).wait()
            pltpu.make_async_copy(dstp.at[wid, bi], idx_d[st], xd).wait()

        def start_g(slot, st, b):
            pltpu.async_copy(table.at[idx_s[st].at[slot]], bufs[b], gsems[b])

        def wait_g(slot, st, b):
            pltpu.make_async_copy(table.at[idx_s[st].at[slot]], bufs[b],
                                  gsems[b]).wait()

        def start_s(slot, st, b):
            pltpu.async_copy(bufs[b], accum.at[idx_d[st].at[slot]], ssems[b],
                             add=True)

        def wait_s(slot, st, b):
            pltpu.make_async_copy(bufs[b], accum.at[idx_d[st].at[slot]],
                                  ssems[b]).wait()

        stage_async(0, 0)
        stage_wait(0, 0)
        stage_async(1, 1)
        for b in range(DEPTH):
            start_g(b, 0, b)
        init.wait()
        plsc.subcore_barrier()

        for k in range(CH):
            blk, slot, st, b = k // BLK, k % BLK, (k // BLK) % 2, k % DEPTH
            wait_g(slot, st, b)
            start_s(slot, st, b)
            wait_s(slot, st, b)
            if slot == BLK - 1 and blk + 2 < NB:
                stage_async(blk + 2, blk % 2)
            kn = k + DEPTH
            if kn < CH:
                bn, sn, stn = kn // BLK, kn % BLK, (kn // BLK) % 2
                if bn != blk and sn == 0:
                    stage_wait(bn, stn)
                start_g(sn, stn, b)

        plsc.subcore_barrier()
        pltpu.sync_copy(accum.at[pl.ds(base, rows_per)],
                        out.at[c, pl.ds(base, rows_per)])

    return conv


_sc_conv_h = _make_sc_conv(H)
_sc_conv_c = _make_sc_conv(CP, tc_tiling=False)


@functools.partial(
    pl.kernel,
    out_type=jax.ShapeDtypeStruct((NC, NACC), jnp.float32),
    mesh=_mesh,
    scratch_types=[
        pltpu.VMEM((BLK, CHUNK), jnp.int32),
        pltpu.VMEM((CHUNK,), jnp.float32),
        pltpu.SemaphoreType.DMA,
        pltpu.VMEM_SHARED((NACC,), jnp.float32),
    ],
)
def _sc_degree(dstp, zeros, out, idx_d, ones, ssem, accum):
    c = lax.axis_index("c")
    s = lax.axis_index("s")
    wid = c * NS + s
    rows_per = NACC // NS
    base = s * rows_per
    pltpu.sync_copy(zeros.at[pl.ds(base, rows_per)],
                    accum.at[pl.ds(base, rows_per)])

    @pl.loop(0, CHUNK // 16)
    def _(i):
        ones[pl.ds(i * 16, 16)] = jnp.ones((16,), jnp.float32)

    plsc.subcore_barrier()

    @pl.loop(0, NB)
    def _(bi):
        pltpu.sync_copy(dstp.at[wid, bi], idx_d)
        for k in range(BLK):
            pltpu.async_copy(ones, accum.at[idx_d.at[k]], ssem, add=True)
        for k in range(BLK):
            pltpu.make_async_copy(ones, accum.at[idx_d.at[k]], ssem).wait()

    plsc.subcore_barrier()
    pltpu.sync_copy(accum.at[pl.ds(base, rows_per)],
                    out.at[c, pl.ds(base, rows_per)])


_BN_SCALE = np.float32(1.0 / math.sqrt(1.0 + 1e-5))


def _tc1_body(degp, x, w, dis, table):
    deg = 1.0 + degp[0][:N] + degp[1][:N]
    d = lax.rsqrt(deg)
    dis[...] = d
    table[...] = jnp.dot(x[...], w[...],
                         preferred_element_type=jnp.float32) * d


_tc1 = pl.pallas_call(
    _tc1_body,
    out_shape=(jax.ShapeDtypeStruct((N, 1), jnp.float32),
               jax.ShapeDtypeStruct((N, H), jnp.float32)),
)


def _tc_mid_body(p, table, dis, b, g, be, w, out):
    acc = p[0][:N] + p[1][:N] + table[...]
    d = dis[...]
    pre = d * acc + b[...]
    h = jnp.maximum(g[...] * (pre * _BN_SCALE) + be[...], 0.0)
    out[...] = jnp.dot(h, w[...], preferred_element_type=jnp.float32) * d


_tc2 = pl.pallas_call(
    _tc_mid_body, out_shape=jax.ShapeDtypeStruct((N, H), jnp.float32))
_tc3 = pl.pallas_call(
    _tc_mid_body, out_shape=jax.ShapeDtypeStruct((N, CP), jnp.float32))


def _tc4_body(p, table, dis, b, out):
    acc = p[0][:N] + p[1][:N] + table[...]
    pre = dis[...] * acc + b[...]
    col = lax.broadcasted_iota(jnp.int32, (N, CP), 1)
    masked = jnp.where(col < C, pre, -jnp.inf)
    m = jnp.max(masked, axis=1, keepdims=True)
    ex = jnp.exp(masked - m)
    res = pre - m - jnp.log(jnp.sum(ex, axis=1, keepdims=True))
    out[...] = res[:, :C]


_tc4 = pl.pallas_call(
    _tc4_body, out_shape=jax.ShapeDtypeStruct((N, C), jnp.float32))


def kernel(x, edge_index, W1, b1, g1, be1, W2, b2, g2, be2, W3, b3):
    src = edge_index[0]
    dst = edge_index[1]
    pad = E_PAD - E
    pad_idx = jnp.arange(pad, dtype=jnp.int32) % NPAD
    srcp = jnp.concatenate([src, pad_idx]).reshape(NW, NB, BLK, CHUNK)
    dstp = jnp.concatenate([dst, N + pad_idx]).reshape(NW, NB, BLK, CHUNK)
    z1 = jnp.zeros((NACC,), jnp.float32)
    zh = jnp.zeros((NACC, H), jnp.float32)
    zc = jnp.zeros((NACC, CP), jnp.float32)

    degp = _sc_degree(dstp, z1)
    dis, t1 = _tc1(degp.reshape(NC, NACC, 1), x, W1)
    p1 = _sc_conv_h(t1, srcp, dstp, zh)
    t2 = _tc2(p1, t1, dis, b1.reshape(1, H), g1.reshape(1, H),
              be1.reshape(1, H), W2)
    p2 = _sc_conv_h(t2, srcp, dstp, zh)
    w3p = jnp.pad(W3, ((0, 0), (0, CP - C)))
    b3p = jnp.pad(b3, (0, CP - C))
    t3 = _tc3(p2, t2, dis, b2.reshape(1, H), g2.reshape(1, H),
              be2.reshape(1, H), w3p)
    p3 = _sc_conv_c(t3, srcp, dstp, zc)
    return _tc4(p3, t3, dis, b3p.reshape(1, CP))
